# speedup vs baseline: 1.7071x; 1.7071x over previous
import functools

import jax
import jax.numpy as jnp
from jax.experimental import pallas as pl
from jax.experimental.pallas import tpu as pltpu

N_CLASSES = 81
TOP_K = 200
CONF_THRESH = 0.01
IOU_THRESH = 0.45
IMG_H = 512.0
IMG_W = 512.0

N = 20000
N_PAD = 20096
ROWS = N_PAD // 128
NEG = -1.0


def _nms_body(y_ref, out_ref, acc_ref):
    conf = y_ref[0, 0]
    for c in range(1, N_CLASSES):
        conf = jnp.maximum(conf, y_ref[0, c])
    cls = jnp.zeros((ROWS, 128), jnp.float32)
    for c in range(N_CLASSES - 1, -1, -1):
        cls = jnp.where(y_ref[0, c] == conf, jnp.float32(c), cls)

    v81 = y_ref[0, 81]
    v82 = y_ref[0, 82]
    v83 = y_ref[0, 83]
    v84 = y_ref[0, 84]
    v85 = y_ref[0, 85]
    v86 = y_ref[0, 86]
    v87 = y_ref[0, 87]
    v88 = y_ref[0, 88]
    v89 = y_ref[0, 89]
    v90 = y_ref[0, 90]
    v91 = y_ref[0, 91]
    v92 = y_ref[0, 92]

    cx = v81 * v89 * v87 + v85
    cy = v82 * v90 * v88 + v86
    w = jnp.exp(v83 * v91) * v87
    h = jnp.exp(v84 * v92) * v88
    x1 = (cx - 0.5 * w) * IMG_W
    y1 = (cy - 0.5 * h) * IMG_H
    x2 = (cx + 0.5 * w) * IMG_W
    y2 = (cy + 0.5 * h) * IMG_H
    areas = jnp.maximum(x2 - x1, 0.0) * jnp.maximum(y2 - y1, 0.0)

    valid = (cls != 0.0) & (conf > CONF_THRESH)
    s0 = jnp.where(valid, conf, NEG)
    iota_r = jax.lax.broadcasted_iota(jnp.int32, (ROWS, 128), 0)
    iota_l = jax.lax.broadcasted_iota(jnp.int32, (ROWS, 128), 1)
    flat_idx = iota_r * 128 + iota_l
    s0 = jnp.where(flat_idx < N, s0, NEG)

    lane_iota = jax.lax.broadcasted_iota(jnp.int32, (1, 128), 1)

    def body(t, s):
        m = jnp.max(s)
        at_max = s == m
        hit = at_max & (flat_idx == jnp.min(jnp.where(at_max, flat_idx, N_PAD)))

        def pick(v):
            return jnp.max(jnp.where(hit, v, -3.4e38))

        sx1 = pick(x1)
        sy1 = pick(y1)
        sx2 = pick(x2)
        sy2 = pick(y2)
        sar = pick(areas)
        scl = pick(cls)
        ok = m > 0.0

        ix1 = jnp.maximum(x1, sx1)
        iy1 = jnp.maximum(y1, sy1)
        ix2 = jnp.minimum(x2, sx2)
        iy2 = jnp.minimum(y2, sy2)
        inter = jnp.maximum(ix2 - ix1, 0.0) * jnp.maximum(iy2 - iy1, 0.0)
        union = jnp.maximum(areas + sar - inter, 1e-9)
        iou = inter / union
        s_new = jnp.where((iou > IOU_THRESH) | hit, NEG, s)

        row = jnp.where(lane_iota == 0, jnp.where(ok, scl, 0.0),
              jnp.where(lane_iota == 1, jnp.where(ok, m, 0.0),
              jnp.where(lane_iota == 2, jnp.where(ok, sx1, 0.0),
              jnp.where(lane_iota == 3, jnp.where(ok, sy1, 0.0),
              jnp.where(lane_iota == 4, jnp.where(ok, sx2, 0.0),
                        jnp.where(ok, sy2, 0.0))))))
        acc_ref[pl.ds(t, 1), :] = row
        return s_new

    jax.lax.fori_loop(0, TOP_K, body, s0)
    out_ref[0] = acc_ref[:, :6]


@jax.jit
def kernel(y_pred):
    B = y_pred.shape[0]
    y = jnp.pad(y_pred, ((0, 0), (0, N_PAD - N), (0, 0)))
    y = jnp.transpose(y, (0, 2, 1)).reshape(B, 93, ROWS, 128)
    out = pl.pallas_call(
        _nms_body,
        grid=(B,),
        in_specs=[pl.BlockSpec((1, 93, ROWS, 128), lambda b: (b, 0, 0, 0))],
        out_specs=pl.BlockSpec((1, TOP_K, 6), lambda b: (b, 0, 0)),
        out_shape=jax.ShapeDtypeStruct((B, TOP_K, 6), jnp.float32),
        scratch_shapes=[pltpu.VMEM((TOP_K, 128), jnp.float32)],
        compiler_params=pltpu.CompilerParams(
            dimension_semantics=("arbitrary",)),
    )(y)
    return out
